# Initial kernel scaffold; baseline (speedup 1.0000x reference)
import jax
import jax.numpy as jnp
from jax import lax
from jax.experimental import pallas as pl
from jax.experimental.pallas import tpu as pltpu
from jax.experimental.pallas import tpu_sc as plsc

N_EDGES = 320000
D_FEAT = 128
N_SEGMENTS = 10000

NUM_CORES = 2
NUM_SUBCORES = 16
NUM_WORKERS = NUM_CORES * NUM_SUBCORES
LANES = 16
NREG = D_FEAT // LANES

SEGS_PER_WORKER = -(-N_SEGMENTS // NUM_WORKERS)
OUT_PAD = SEGS_PER_WORKER * NUM_WORKERS
CHUNK = 200
NUM_CHUNKS = N_EDGES // CHUNK


def _body(feats_hbm, batch_hbm, parms_hbm, out_hbm, pv, bbuf, fbuf, local):
    wid = lax.axis_index("s") * NUM_CORES + lax.axis_index("c")
    pltpu.sync_copy(parms_hbm, pv)
    c0 = pv[wid]
    c1 = pv[NUM_WORKERS + wid]
    s_lo = wid * SEGS_PER_WORKER

    ones = jnp.ones((LANES,), jnp.float32)

    def init_row(i, carry):
        for k in range(NREG):
            local[i, pl.ds(LANES * k, LANES)] = ones
        return carry

    lax.fori_loop(0, SEGS_PER_WORKER + 2, init_row, 0)

    def chunk_body(c, carry):
        accs, b_prev = carry
        pltpu.sync_copy(batch_hbm.at[pl.ds(c * CHUNK, CHUNK)], bbuf)
        pltpu.sync_copy(feats_hbm.at[pl.ds(c * CHUNK, CHUNK)], fbuf)

        def row_body(j, rc):
            raccs, rb_prev = rc
            b = bbuf[j]
            same = jnp.broadcast_to(b, (LANES,)) == jnp.broadcast_to(
                rb_prev, (LANES,))
            idx = jnp.clip(b - s_lo + 1, 0, SEGS_PER_WORKER + 1)
            new_accs = []
            for k in range(NREG):
                row = fbuf[j, pl.ds(LANES * k, LANES)]
                a = jnp.where(same, raccs[k] * row, row)
                local[idx, pl.ds(LANES * k, LANES)] = a
                new_accs.append(a)
            return tuple(new_accs), b

        return lax.fori_loop(0, CHUNK, row_body, (accs, b_prev))

    init = (tuple(ones for _ in range(NREG)), jnp.int32(-1))
    lax.fori_loop(c0, c1, chunk_body, init)

    pltpu.sync_copy(
        local.at[pl.ds(1, SEGS_PER_WORKER)],
        out_hbm.at[pl.ds(s_lo, SEGS_PER_WORKER)],
    )


_sc_call = pl.kernel(
    _body,
    out_type=jax.ShapeDtypeStruct((OUT_PAD, D_FEAT), jnp.float32),
    mesh=plsc.VectorSubcoreMesh(core_axis_name="c", subcore_axis_name="s"),
    scratch_types=[
        pltpu.VMEM((2 * NUM_WORKERS,), jnp.int32),
        pltpu.VMEM((CHUNK,), jnp.int32),
        pltpu.VMEM((CHUNK, D_FEAT), jnp.float32),
        pltpu.VMEM((SEGS_PER_WORKER + 2, D_FEAT), jnp.float32),
    ],
)


@jax.jit
def kernel(feats, batch):
    batch = batch.astype(jnp.int32)
    seg_bounds = jnp.arange(NUM_WORKERS + 1, dtype=jnp.int32) * SEGS_PER_WORKER
    row_bounds = jnp.searchsorted(batch, seg_bounds, side="left").astype(
        jnp.int32)
    c0 = row_bounds[:-1] // CHUNK
    c1 = -(-row_bounds[1:] // CHUNK)
    parms = jnp.concatenate([c0, c1]).astype(jnp.int32)
    out = _sc_call(feats, batch, parms)
    return out[:N_SEGMENTS]


# speedup vs baseline: 2.9974x; 2.9974x over previous
import jax
import jax.numpy as jnp
from jax import lax
from jax.experimental import pallas as pl
from jax.experimental.pallas import tpu as pltpu
from jax.experimental.pallas import tpu_sc as plsc

N_EDGES = 320000
D_FEAT = 128
N_SEGMENTS = 10000

NUM_CORES = 2
NUM_SUBCORES = 16
NUM_WORKERS = NUM_CORES * NUM_SUBCORES
LANES = 16
NREG = D_FEAT // LANES

SEGS_PER_WORKER = 320
OUT_PAD = SEGS_PER_WORKER * NUM_WORKERS
CHUNK = 200
NUM_CHUNKS = N_EDGES // CHUNK


def _body(feats_hbm, batch_hbm, parms_hbm, out_hbm, pv, bbuf, fbuf, local):
    wid = lax.axis_index("s") * NUM_CORES + lax.axis_index("c")
    pltpu.sync_copy(parms_hbm, pv)
    c0 = pv[pl.ds(wid, LANES)][0]
    c1 = pv[pl.ds(NUM_WORKERS + wid, LANES)][0]
    s_lo = wid * SEGS_PER_WORKER

    ones = jnp.ones((LANES,), jnp.float32)

    def init_row(i, carry):
        for k in range(NREG):
            local[i, pl.ds(LANES * k, LANES)] = ones
        return carry

    lax.fori_loop(0, SEGS_PER_WORKER + 1, init_row, 0)

    def chunk_body(c, carry):
        accs, b_prev = carry
        pltpu.sync_copy(batch_hbm.at[pl.ds(c * CHUNK, CHUNK)],
                        bbuf.at[pl.ds(0, CHUNK)])
        pltpu.sync_copy(feats_hbm.at[pl.ds(c * CHUNK, CHUNK)], fbuf)

        def row_body(j, rc):
            raccs, rb_prev = rc
            b = bbuf[pl.ds(j, LANES)][0]
            f = jnp.where(b == rb_prev, jnp.float32(1.0), jnp.float32(0.0))
            fv = jnp.broadcast_to(f, (LANES,))
            omv = jnp.broadcast_to(jnp.float32(1.0) - f, (LANES,))
            rel = b - s_lo
            oob = (rel < 0) | (rel >= SEGS_PER_WORKER)
            idx = jnp.where(oob, SEGS_PER_WORKER, rel)
            new_accs = []
            for k in range(NREG):
                row = fbuf[j, pl.ds(LANES * k, LANES)]
                a = row * (raccs[k] * fv + omv)
                local[idx, pl.ds(LANES * k, LANES)] = a
                new_accs.append(a)
            return tuple(new_accs), b

        return lax.fori_loop(0, CHUNK, row_body, (accs, b_prev))

    init = (tuple(ones for _ in range(NREG)), jnp.int32(-1))
    lax.fori_loop(c0, c1, chunk_body, init)

    pltpu.sync_copy(
        local.at[pl.ds(0, SEGS_PER_WORKER)],
        out_hbm.at[pl.ds(s_lo, SEGS_PER_WORKER)],
    )


_sc_call = pl.kernel(
    _body,
    out_type=jax.ShapeDtypeStruct((OUT_PAD, D_FEAT), jnp.float32),
    mesh=plsc.VectorSubcoreMesh(core_axis_name="c", subcore_axis_name="s"),
    scratch_types=[
        pltpu.VMEM((2 * NUM_WORKERS + LANES,), jnp.int32),
        pltpu.VMEM((CHUNK + LANES,), jnp.int32),
        pltpu.VMEM((CHUNK, D_FEAT), jnp.float32),
        pltpu.VMEM((SEGS_PER_WORKER + 8, D_FEAT), jnp.float32),
    ],
)


@jax.jit
def kernel(feats, batch):
    batch = batch.astype(jnp.int32)
    seg_bounds = jnp.arange(NUM_WORKERS + 1, dtype=jnp.int32) * SEGS_PER_WORKER
    row_bounds = jnp.searchsorted(batch, seg_bounds, side="left").astype(
        jnp.int32)
    c0 = row_bounds[:-1] // CHUNK
    c1 = -(-row_bounds[1:] // CHUNK)
    parms = jnp.concatenate(
        [c0, c1, jnp.zeros((LANES,), jnp.int32)]).astype(jnp.int32)
    out = _sc_call(feats, batch, parms)
    return out[:N_SEGMENTS]
